# baseline (device time: 116304 ns/iter reference)
import jax
import jax.numpy as jnp
from jax import lax
from jax.experimental import pallas as pl
from jax.experimental.pallas import tpu as pltpu

N_DEV = 32
PLANE = 8
NZ = 4
SQ = 1024
DM = 1024
CHUNK = DM // PLANE
SUB = CHUNK // NZ
DH = 128

_MESH = pl.DeviceIdType.MESH
_BF = jnp.bfloat16
_F32 = jnp.float32


def _allreduce_body(p_ref, out_ref,
                    comm_cw, comm_ccw, stage_cw, stage_ccw,
                    zcomm, zbcast, stage_b1, stage_b2,
                    p3cw, p3ccw,
                    send_cw, recv_cw, send_ccw, recv_ccw,
                    send_b1, recv_b1, send_b2, recv_b2,
                    send_p3cw, recv_p3cw, send_p3ccw, recv_p3ccw):
    my = lax.axis_index("i")
    q = lax.rem(my, PLANE)
    zi = my // PLANE
    base = my - q
    pright = base + lax.rem(q + 1, PLANE)
    pleft = base + lax.rem(q + PLANE - 1, PLANE)

    barrier_sem = pltpu.get_barrier_semaphore()
    peers = [pleft, pright] + [
        lax.rem(zi + d, NZ) * PLANE + q for d in range(1, NZ)
    ]
    for pr in peers:
        pl.semaphore_signal(barrier_sem, inc=1,
                            device_id=(pr,), device_id_type=_MESH)
    pl.semaphore_wait(barrier_sem, len(peers))

    out_ref[...] = p_ref[...]

    for s in range(4):
        rows_s = pl.ds(lax.rem(q + 5 - s + PLANE, PLANE) * CHUNK, CHUNK)
        stage_cw[...] = out_ref[rows_s, :].astype(_BF)
        rdma_cw = pltpu.make_async_remote_copy(
            src_ref=stage_cw,
            dst_ref=comm_cw.at[s],
            send_sem=send_cw,
            recv_sem=recv_cw.at[s],
            device_id=(pright,),
            device_id_type=_MESH,
        )
        rdma_cw.start()
        if s < 3:
            rows_t = pl.ds(lax.rem(q + 6 + s, PLANE) * CHUNK, CHUNK)
            stage_ccw[...] = out_ref[rows_t, :].astype(_BF)
            rdma_ccw = pltpu.make_async_remote_copy(
                src_ref=stage_ccw,
                dst_ref=comm_ccw.at[s],
                send_sem=send_ccw,
                recv_sem=recv_ccw.at[s],
                device_id=(pleft,),
                device_id_type=_MESH,
            )
            rdma_ccw.start()
        rdma_cw.wait()
        rows = pl.ds(lax.rem(q + 4 - s + PLANE, PLANE) * CHUNK, CHUNK)
        out_ref[rows, :] = out_ref[rows, :] + comm_cw[s].astype(_F32)
        if s < 3:
            rdma_ccw.wait()
            rows = pl.ds(lax.rem(q + 7 + s, PLANE) * CHUNK, CHUNK)
            out_ref[rows, :] = out_ref[rows, :] + comm_ccw[s].astype(_F32)

    c_own = lax.rem(q + 1, PLANE)
    slab = c_own * CHUNK

    b1 = []
    for d in range(1, NZ):
        tz = lax.rem(zi + d, NZ)
        tgt = tz * PLANE + q
        stage_b1[d - 1, :, :] = out_ref[pl.ds(slab + tz * SUB, SUB), :].astype(_BF)
        rdma = pltpu.make_async_remote_copy(
            src_ref=stage_b1.at[d - 1],
            dst_ref=zcomm.at[zi],
            send_sem=send_b1.at[d - 1],
            recv_sem=recv_b1.at[zi],
            device_id=(tgt,),
            device_id_type=_MESH,
        )
        rdma.start()
        b1.append(rdma)
    for d in range(1, NZ):
        sz = lax.rem(zi + d, NZ)
        rw = pltpu.make_async_remote_copy(
            src_ref=zcomm.at[sz],
            dst_ref=zcomm.at[sz],
            send_sem=send_b1.at[d - 1],
            recv_sem=recv_b1.at[sz],
            device_id=(my,),
            device_id_type=_MESH,
        )
        rw.wait_recv()
    myrows = pl.ds(slab + zi * SUB, SUB)
    acc = out_ref[myrows, :]
    for d in range(1, NZ):
        sz = lax.rem(zi + d, NZ)
        acc = acc + zcomm[sz].astype(_F32)
    out_ref[myrows, :] = acc
    stage_b2[...] = acc.astype(_BF)
    for rdma in b1:
        rdma.wait_send()

    b2 = []
    for d in range(1, NZ):
        tz = lax.rem(zi + d, NZ)
        tgt = tz * PLANE + q
        rdma = pltpu.make_async_remote_copy(
            src_ref=stage_b2,
            dst_ref=zbcast.at[zi],
            send_sem=send_b2.at[d - 1],
            recv_sem=recv_b2.at[zi],
            device_id=(tgt,),
            device_id_type=_MESH,
        )
        rdma.start()
        b2.append(rdma)
    for d in range(1, NZ):
        sz = lax.rem(zi + d, NZ)
        rw = pltpu.make_async_remote_copy(
            src_ref=zbcast.at[sz],
            dst_ref=zbcast.at[sz],
            send_sem=send_b2.at[d - 1],
            recv_sem=recv_b2.at[sz],
            device_id=(my,),
            device_id_type=_MESH,
        )
        rw.wait_recv()
        out_ref[pl.ds(slab + sz * SUB, SUB), :] = zbcast[sz].astype(_F32)
    for rdma in b2:
        rdma.wait_send()

    stage_cw[...] = out_ref[pl.ds(slab, CHUNK), :].astype(_BF)
    for s in range(4):
        rdma_cw = pltpu.make_async_remote_copy(
            src_ref=stage_cw if s == 0 else p3cw.at[s - 1],
            dst_ref=p3cw.at[s],
            send_sem=send_p3cw,
            recv_sem=recv_p3cw.at[s],
            device_id=(pright,),
            device_id_type=_MESH,
        )
        rdma_cw.start()
        if s < 3:
            rdma_ccw = pltpu.make_async_remote_copy(
                src_ref=stage_cw if s == 0 else p3ccw.at[s - 1],
                dst_ref=p3ccw.at[s],
                send_sem=send_p3ccw,
                recv_sem=recv_p3ccw.at[s],
                device_id=(pleft,),
                device_id_type=_MESH,
            )
            rdma_ccw.start()
        rdma_cw.wait()
        rows = pl.ds(lax.rem(q + PLANE - s, PLANE) * CHUNK, CHUNK)
        out_ref[rows, :] = p3cw[s].astype(_F32)
        if s < 3:
            rdma_ccw.wait()
            rows = pl.ds(lax.rem(q + 2 + s, PLANE) * CHUNK, CHUNK)
            out_ref[rows, :] = p3ccw[s].astype(_F32)


def _ring_allreduce(partial):
    return pl.pallas_call(
        _allreduce_body,
        out_shape=jax.ShapeDtypeStruct((DM, DM), jnp.float32),
        in_specs=[pl.BlockSpec(memory_space=pltpu.VMEM)],
        out_specs=pl.BlockSpec(memory_space=pltpu.VMEM),
        scratch_shapes=[
            pltpu.VMEM((4, CHUNK, DM), _BF),
            pltpu.VMEM((3, CHUNK, DM), _BF),
            pltpu.VMEM((CHUNK, DM), _BF),
            pltpu.VMEM((CHUNK, DM), _BF),
            pltpu.VMEM((NZ, SUB, DM), _BF),
            pltpu.VMEM((NZ, SUB, DM), _BF),
            pltpu.VMEM((NZ - 1, SUB, DM), _BF),
            pltpu.VMEM((SUB, DM), _BF),
            pltpu.VMEM((4, CHUNK, DM), _BF),
            pltpu.VMEM((3, CHUNK, DM), _BF),
            pltpu.SemaphoreType.DMA,
            pltpu.SemaphoreType.DMA((4,)),
            pltpu.SemaphoreType.DMA,
            pltpu.SemaphoreType.DMA((3,)),
            pltpu.SemaphoreType.DMA((NZ - 1,)),
            pltpu.SemaphoreType.DMA((NZ,)),
            pltpu.SemaphoreType.DMA((NZ - 1,)),
            pltpu.SemaphoreType.DMA((NZ,)),
            pltpu.SemaphoreType.DMA,
            pltpu.SemaphoreType.DMA((4,)),
            pltpu.SemaphoreType.DMA,
            pltpu.SemaphoreType.DMA((3,)),
        ],
        compiler_params=pltpu.CompilerParams(collective_id=0),
    )(partial)


def kernel(x, Wq, K_ext, V_ext, Wo):
    i = lax.axis_index("i")
    hs = Wq.shape[1] // DH
    scale = 0.08838834764831843

    xb = x[0].astype(jnp.bfloat16)
    qall = jnp.dot(xb, Wq.astype(jnp.bfloat16),
                   preferred_element_type=jnp.float32)
    qall = qall.reshape(SQ, hs, DH).astype(jnp.bfloat16)

    k = lax.dynamic_slice_in_dim(K_ext[0], i * hs, hs, axis=1)
    v = lax.dynamic_slice_in_dim(V_ext[0], i * hs, hs, axis=1)
    k = k.astype(jnp.bfloat16)
    v = v.astype(jnp.bfloat16)

    BQ, W, NB, G = 256, 512, SQ // 256, 32
    k0s = [min(max(BQ * b - 128, 0), SQ - W) for b in range(NB)]
    kw = jnp.stack([lax.slice_in_dim(k, k0, k0 + W, axis=0) for k0 in k0s])
    vw = jnp.stack([lax.slice_in_dim(v, k0, k0 + W, axis=0) for k0 in k0s])
    kcat = jnp.concatenate(
        [jnp.broadcast_to(k[None, :G], (NB, G, hs, DH)), kw], axis=1)
    vcat = jnp.concatenate(
        [jnp.broadcast_to(v[None, :G], (NB, G, hs, DH)), vw], axis=1)
    qb = qall.reshape(NB, BQ, hs, DH)

    s = jnp.einsum("bihd,bjhd->bhij", qb, kcat,
                   preferred_element_type=jnp.float32) * scale
    qi = (jnp.arange(NB) * BQ)[:, None, None] + jnp.arange(BQ)[None, :, None]
    kiw = jnp.asarray(k0s)[:, None] + jnp.arange(W)[None, :]
    live_w = ((jnp.abs(qi - kiw[:, None, :]) <= 128)
              | (kiw[:, None, :] < G))
    live_g = jnp.broadcast_to(
        (jnp.arange(NB) > 0)[:, None, None], (NB, BQ, G))
    live = jnp.concatenate([live_g, live_w], axis=-1)
    s = jnp.where(live[:, None], s, -1e9)

    wts = jnp.exp(s)
    wts = wts / wts.sum(axis=-1, keepdims=True)
    ctx_b = jnp.einsum("bhij,bjhd->bihd", wts.astype(jnp.bfloat16), vcat,
                       preferred_element_type=jnp.float32)
    ctx_b = ctx_b.reshape(SQ, hs, DH)

    s0 = jnp.einsum("ihd,jhd->hij", qall[:G], k,
                    preferred_element_type=jnp.float32) * scale
    w0 = jnp.exp(s0)
    w0 = w0 / w0.sum(axis=-1, keepdims=True)
    ctx0 = jnp.einsum("hij,jhd->ihd", w0.astype(jnp.bfloat16), v,
                      preferred_element_type=jnp.float32)

    ctx = jnp.concatenate([ctx0, ctx_b[G:]], axis=0)
    ctx = ctx.reshape(SQ, hs * DH).astype(jnp.bfloat16)

    partial = jnp.dot(ctx, Wo.astype(jnp.bfloat16),
                      preferred_element_type=jnp.float32)

    out = _ring_allreduce(partial)
    return out.reshape(1, SQ, DM)


# device time: 111117 ns/iter; 1.0467x vs baseline; 1.0467x over previous
import jax
import jax.numpy as jnp
from jax import lax
from jax.experimental import pallas as pl
from jax.experimental.pallas import tpu as pltpu

N_DEV = 32
PLANE = 8
NZ = 4
SQ = 1024
DM = 1024
CHUNK = DM // PLANE
SUB = CHUNK // NZ
DH = 128

_MESH = pl.DeviceIdType.MESH
_BF = jnp.bfloat16
_F32 = jnp.float32


def _allreduce_body(ctx_ref, wo_ref, out_ref,
                    comm_cw, comm_ccw, stage_cw, stage_ccw,
                    zcomm, zbcast, stage_b1, stage_b2,
                    p3cw, p3ccw,
                    send_cw, recv_cw, send_ccw, recv_ccw,
                    send_b1, recv_b1, send_b2, recv_b2,
                    send_p3cw, recv_p3cw, send_p3ccw, recv_p3ccw):
    my = lax.axis_index("i")
    q = lax.rem(my, PLANE)
    zi = my // PLANE
    base = my - q
    pright = base + lax.rem(q + 1, PLANE)
    pleft = base + lax.rem(q + PLANE - 1, PLANE)

    barrier_sem = pltpu.get_barrier_semaphore()
    peers = [pleft, pright] + [
        lax.rem(zi + d, NZ) * PLANE + q for d in range(1, NZ)
    ]
    for pr in peers:
        pl.semaphore_signal(barrier_sem, inc=1,
                            device_id=(pr,), device_id_type=_MESH)
    pl.semaphore_wait(barrier_sem, len(peers))

    def compute_chunk(c0):
        rows = pl.ds(lax.rem(q + c0, PLANE) * CHUNK, CHUNK)
        out_ref[rows, :] = jnp.dot(ctx_ref[rows, :], wo_ref[...],
                                   preferred_element_type=_F32)

    def start_cw(s):
        rows = pl.ds(lax.rem(q + 5 - s + PLANE, PLANE) * CHUNK, CHUNK)
        stage_cw[...] = out_ref[rows, :].astype(_BF)
        r = pltpu.make_async_remote_copy(
            src_ref=stage_cw,
            dst_ref=comm_cw.at[s],
            send_sem=send_cw,
            recv_sem=recv_cw.at[s],
            device_id=(pright,),
            device_id_type=_MESH,
        )
        r.start()
        return r

    def start_ccw(s):
        rows = pl.ds(lax.rem(q + 6 + s, PLANE) * CHUNK, CHUNK)
        stage_ccw[...] = out_ref[rows, :].astype(_BF)
        r = pltpu.make_async_remote_copy(
            src_ref=stage_ccw,
            dst_ref=comm_ccw.at[s],
            send_sem=send_ccw,
            recv_sem=recv_ccw.at[s],
            device_id=(pleft,),
            device_id_type=_MESH,
        )
        r.start()
        return r

    compute_chunk(5)
    compute_chunk(6)
    rdma_cw = start_cw(0)
    rdma_ccw = start_ccw(0)
    for c0 in (4, 7, 3, 0, 2, 1):
        compute_chunk(c0)

    for s in range(4):
        rdma_cw.wait()
        rows = pl.ds(lax.rem(q + 4 - s + PLANE, PLANE) * CHUNK, CHUNK)
        out_ref[rows, :] = out_ref[rows, :] + comm_cw[s].astype(_F32)
        if s < 3:
            rdma_ccw.wait()
            rows = pl.ds(lax.rem(q + 7 + s, PLANE) * CHUNK, CHUNK)
            out_ref[rows, :] = out_ref[rows, :] + comm_ccw[s].astype(_F32)
            rdma_cw = start_cw(s + 1)
            if s < 2:
                rdma_ccw = start_ccw(s + 1)

    c_own = lax.rem(q + 1, PLANE)
    slab = c_own * CHUNK

    b1 = []
    for d in range(1, NZ):
        tz = lax.rem(zi + d, NZ)
        tgt = tz * PLANE + q
        stage_b1[d - 1, :, :] = out_ref[pl.ds(slab + tz * SUB, SUB), :].astype(_BF)
        rdma = pltpu.make_async_remote_copy(
            src_ref=stage_b1.at[d - 1],
            dst_ref=zcomm.at[zi],
            send_sem=send_b1.at[d - 1],
            recv_sem=recv_b1.at[zi],
            device_id=(tgt,),
            device_id_type=_MESH,
        )
        rdma.start()
        b1.append(rdma)
    for d in range(1, NZ):
        sz = lax.rem(zi + d, NZ)
        rw = pltpu.make_async_remote_copy(
            src_ref=zcomm.at[sz],
            dst_ref=zcomm.at[sz],
            send_sem=send_b1.at[d - 1],
            recv_sem=recv_b1.at[sz],
            device_id=(my,),
            device_id_type=_MESH,
        )
        rw.wait_recv()
    myrows = pl.ds(slab + zi * SUB, SUB)
    acc = out_ref[myrows, :]
    for d in range(1, NZ):
        sz = lax.rem(zi + d, NZ)
        acc = acc + zcomm[sz].astype(_F32)
    out_ref[myrows, :] = acc
    stage_b2[...] = acc.astype(_BF)
    for rdma in b1:
        rdma.wait_send()

    b2 = []
    for d in range(1, NZ):
        tz = lax.rem(zi + d, NZ)
        tgt = tz * PLANE + q
        rdma = pltpu.make_async_remote_copy(
            src_ref=stage_b2,
            dst_ref=zbcast.at[zi],
            send_sem=send_b2.at[d - 1],
            recv_sem=recv_b2.at[zi],
            device_id=(tgt,),
            device_id_type=_MESH,
        )
        rdma.start()
        b2.append(rdma)
    for d in range(1, NZ):
        sz = lax.rem(zi + d, NZ)
        rw = pltpu.make_async_remote_copy(
            src_ref=zbcast.at[sz],
            dst_ref=zbcast.at[sz],
            send_sem=send_b2.at[d - 1],
            recv_sem=recv_b2.at[sz],
            device_id=(my,),
            device_id_type=_MESH,
        )
        rw.wait_recv()
        out_ref[pl.ds(slab + sz * SUB, SUB), :] = zbcast[sz].astype(_F32)
    for rdma in b2:
        rdma.wait_send()

    stage_cw[...] = out_ref[pl.ds(slab, CHUNK), :].astype(_BF)
    for s in range(4):
        rdma_cw = pltpu.make_async_remote_copy(
            src_ref=stage_cw if s == 0 else p3cw.at[s - 1],
            dst_ref=p3cw.at[s],
            send_sem=send_p3cw,
            recv_sem=recv_p3cw.at[s],
            device_id=(pright,),
            device_id_type=_MESH,
        )
        rdma_cw.start()
        if s < 3:
            rdma_ccw = pltpu.make_async_remote_copy(
                src_ref=stage_cw if s == 0 else p3ccw.at[s - 1],
                dst_ref=p3ccw.at[s],
                send_sem=send_p3ccw,
                recv_sem=recv_p3ccw.at[s],
                device_id=(pleft,),
                device_id_type=_MESH,
            )
            rdma_ccw.start()
        rdma_cw.wait()
        rows = pl.ds(lax.rem(q + PLANE - s, PLANE) * CHUNK, CHUNK)
        out_ref[rows, :] = p3cw[s].astype(_F32)
        if s < 3:
            rdma_ccw.wait()
            rows = pl.ds(lax.rem(q + 2 + s, PLANE) * CHUNK, CHUNK)
            out_ref[rows, :] = p3ccw[s].astype(_F32)


def _ring_allreduce(ctx_bf, wo_bf):
    return pl.pallas_call(
        _allreduce_body,
        out_shape=jax.ShapeDtypeStruct((DM, DM), jnp.float32),
        in_specs=[pl.BlockSpec(memory_space=pltpu.VMEM),
                  pl.BlockSpec(memory_space=pltpu.VMEM)],
        out_specs=pl.BlockSpec(memory_space=pltpu.VMEM),
        scratch_shapes=[
            pltpu.VMEM((4, CHUNK, DM), _BF),
            pltpu.VMEM((3, CHUNK, DM), _BF),
            pltpu.VMEM((CHUNK, DM), _BF),
            pltpu.VMEM((CHUNK, DM), _BF),
            pltpu.VMEM((NZ, SUB, DM), _BF),
            pltpu.VMEM((NZ, SUB, DM), _BF),
            pltpu.VMEM((NZ - 1, SUB, DM), _BF),
            pltpu.VMEM((SUB, DM), _BF),
            pltpu.VMEM((4, CHUNK, DM), _BF),
            pltpu.VMEM((3, CHUNK, DM), _BF),
            pltpu.SemaphoreType.DMA,
            pltpu.SemaphoreType.DMA((4,)),
            pltpu.SemaphoreType.DMA,
            pltpu.SemaphoreType.DMA((3,)),
            pltpu.SemaphoreType.DMA((NZ - 1,)),
            pltpu.SemaphoreType.DMA((NZ,)),
            pltpu.SemaphoreType.DMA((NZ - 1,)),
            pltpu.SemaphoreType.DMA((NZ,)),
            pltpu.SemaphoreType.DMA,
            pltpu.SemaphoreType.DMA((4,)),
            pltpu.SemaphoreType.DMA,
            pltpu.SemaphoreType.DMA((3,)),
        ],
        compiler_params=pltpu.CompilerParams(collective_id=0),
    )(ctx_bf, wo_bf)


def kernel(x, Wq, K_ext, V_ext, Wo):
    i = lax.axis_index("i")
    hs = Wq.shape[1] // DH
    scale = 0.08838834764831843

    xb = x[0].astype(jnp.bfloat16)
    qall = jnp.dot(xb, Wq.astype(jnp.bfloat16),
                   preferred_element_type=jnp.float32)
    qall = qall.reshape(SQ, hs, DH).astype(jnp.bfloat16)

    k = lax.dynamic_slice_in_dim(K_ext[0], i * hs, hs, axis=1)
    v = lax.dynamic_slice_in_dim(V_ext[0], i * hs, hs, axis=1)
    k = k.astype(jnp.bfloat16)
    v = v.astype(jnp.bfloat16)

    BQ, W = 256, 512
    ctx_blocks = []
    for b in range(SQ // BQ):
        k0 = min(max(BQ * b - 128, 0), SQ - W)
        qb = lax.slice_in_dim(qall, b * BQ, (b + 1) * BQ, axis=0)
        kw = lax.slice_in_dim(k, k0, k0 + W, axis=0)
        vw = lax.slice_in_dim(v, k0, k0 + W, axis=0)
        sw = jnp.einsum("ihd,jhd->hij", qb, kw,
                        preferred_element_type=jnp.float32) * scale
        qi = (b * BQ + jnp.arange(BQ))[:, None]
        ki = (k0 + jnp.arange(W))[None, :]
        live = (jnp.abs(qi - ki) <= 128) | (ki < 32)
        sw = jnp.where(live[None], sw, -1e9)
        if b == 0:
            s, vv = sw, vw
        else:
            sg = jnp.einsum("ihd,jhd->hij", qb, k[:32],
                            preferred_element_type=jnp.float32) * scale
            s = jnp.concatenate([sg, sw], axis=-1)
            vv = jnp.concatenate([v[:32], vw], axis=0)
        wts = jnp.exp(s)
        wts = wts / wts.sum(axis=-1, keepdims=True)
        ctx_blocks.append(
            jnp.einsum("hij,jhd->ihd", wts.astype(jnp.bfloat16), vv,
                       preferred_element_type=jnp.float32))

    s0 = jnp.einsum("ihd,jhd->hij", qall[:32], k,
                    preferred_element_type=jnp.float32) * scale
    w0 = jnp.exp(s0)
    w0 = w0 / w0.sum(axis=-1, keepdims=True)
    ctx0 = jnp.einsum("hij,jhd->ihd", w0.astype(jnp.bfloat16), v,
                      preferred_element_type=jnp.float32)

    ctx = jnp.concatenate([ctx0, ctx_blocks[0][32:]] + ctx_blocks[1:],
                          axis=0)
    ctx = ctx.reshape(SQ, hs * DH).astype(jnp.bfloat16)

    out = _ring_allreduce(ctx, Wo.astype(jnp.bfloat16))
    return out.reshape(1, SQ, DM)


# device time: 90619 ns/iter; 1.2834x vs baseline; 1.2262x over previous
import jax
import jax.numpy as jnp
from jax import lax
from jax.experimental import pallas as pl
from jax.experimental.pallas import tpu as pltpu

N_DEV = 32
PLANE = 8
NZ = 4
SQ = 1024
DM = 1024
CHUNK = DM // PLANE
SUB = CHUNK // NZ
DH = 128

_MESH = pl.DeviceIdType.MESH
_BF = jnp.bfloat16
_F32 = jnp.float32


def _allreduce_body(q_ref, k_ref, v_ref, wo_ref, p0_ref, out_ref,
                    ctx_chunk,
                    comm_cw, comm_ccw, stage_cw, stage_ccw,
                    zcomm, zbcast, stage_b1, stage_b2,
                    p3cw, p3ccw,
                    send_cw, recv_cw, send_ccw, recv_ccw,
                    send_b1, recv_b1, send_b2, recv_b2,
                    send_p3cw, recv_p3cw, send_p3ccw, recv_p3ccw):
    my = lax.axis_index("i")
    q = lax.rem(my, PLANE)
    zi = my // PLANE
    base = my - q
    pright = base + lax.rem(q + 1, PLANE)
    pleft = base + lax.rem(q + PLANE - 1, PLANE)

    barrier_sem = pltpu.get_barrier_semaphore()
    peers = [pleft, pright] + [
        lax.rem(zi + d, NZ) * PLANE + q for d in range(1, NZ)
    ]
    for pr in peers:
        pl.semaphore_signal(barrier_sem, inc=1,
                            device_id=(pr,), device_id_type=_MESH)
    pl.semaphore_wait(barrier_sem, len(peers))

    scale = 0.08838834764831843
    HS = 8
    W = 512

    def compute_chunk(c0):
        c = lax.rem(q + c0, PLANE)
        rows = pl.ds(c * CHUNK, CHUNK)

        @pl.when(c == 0)
        def _():
            out_ref[rows, :] = p0_ref[...]

        @pl.when(c != 0)
        def _():
            k0 = jnp.minimum(c * CHUNK - CHUNK, SQ - W)
            qi = c * CHUNK + lax.broadcasted_iota(jnp.int32, (CHUNK, W), 0)
            ki = k0 + lax.broadcasted_iota(jnp.int32, (CHUNK, W), 1)
            livew = (jnp.abs(qi - ki) <= 128) | (ki < 32)
            for h in range(HS):
                qh = q_ref[h, rows, :]
                kw = k_ref[h, pl.ds(k0, W), :]
                sw = lax.dot_general(
                    qh, kw, (((1,), (1,)), ((), ())),
                    preferred_element_type=_F32) * scale
                ew = jnp.where(livew, jnp.exp(sw), 0.0)
                sg = lax.dot_general(
                    qh, k_ref[h, 0:32, :], (((1,), (1,)), ((), ())),
                    preferred_element_type=_F32) * scale
                eg = jnp.where(k0 > 0, jnp.exp(sg), 0.0)
                denom = (ew.sum(axis=1, keepdims=True)
                         + eg.sum(axis=1, keepdims=True))
                ww = (ew / denom).astype(_BF)
                wg = (eg / denom).astype(_BF)
                ctx_h = (jnp.dot(ww, v_ref[h, pl.ds(k0, W), :],
                                 preferred_element_type=_F32)
                         + jnp.dot(wg, v_ref[h, 0:32, :],
                                   preferred_element_type=_F32))
                ctx_chunk[:, h * DH:(h + 1) * DH] = ctx_h.astype(_BF)
            out_ref[rows, :] = jnp.dot(ctx_chunk[...], wo_ref[...],
                                       preferred_element_type=_F32)

    def start_cw(s):
        rows = pl.ds(lax.rem(q + 5 - s + PLANE, PLANE) * CHUNK, CHUNK)
        stage_cw[...] = out_ref[rows, :].astype(_BF)
        r = pltpu.make_async_remote_copy(
            src_ref=stage_cw,
            dst_ref=comm_cw.at[s],
            send_sem=send_cw,
            recv_sem=recv_cw.at[s],
            device_id=(pright,),
            device_id_type=_MESH,
        )
        r.start()
        return r

    def start_ccw(s):
        rows = pl.ds(lax.rem(q + 6 + s, PLANE) * CHUNK, CHUNK)
        stage_ccw[...] = out_ref[rows, :].astype(_BF)
        r = pltpu.make_async_remote_copy(
            src_ref=stage_ccw,
            dst_ref=comm_ccw.at[s],
            send_sem=send_ccw,
            recv_sem=recv_ccw.at[s],
            device_id=(pleft,),
            device_id_type=_MESH,
        )
        r.start()
        return r

    compute_chunk(5)
    rdma_cw = start_cw(0)
    compute_chunk(6)
    rdma_ccw = start_ccw(0)
    compute_chunk(4)
    compute_chunk(7)

    for s in range(4):
        rdma_cw.wait()
        rows = pl.ds(lax.rem(q + 4 - s + PLANE, PLANE) * CHUNK, CHUNK)
        out_ref[rows, :] = out_ref[rows, :] + comm_cw[s].astype(_F32)
        if s < 3:
            rdma_ccw.wait()
            rows = pl.ds(lax.rem(q + 7 + s, PLANE) * CHUNK, CHUNK)
            out_ref[rows, :] = out_ref[rows, :] + comm_ccw[s].astype(_F32)
            rdma_cw = start_cw(s + 1)
            if s < 2:
                rdma_ccw = start_ccw(s + 1)
        if s == 0:
            compute_chunk(3)
            compute_chunk(0)
        elif s == 1:
            compute_chunk(2)
            compute_chunk(1)

    c_own = lax.rem(q + 1, PLANE)
    slab = c_own * CHUNK

    b1 = []
    for d in range(1, NZ):
        tz = lax.rem(zi + d, NZ)
        tgt = tz * PLANE + q
        stage_b1[d - 1, :, :] = out_ref[pl.ds(slab + tz * SUB, SUB), :].astype(_BF)
        rdma = pltpu.make_async_remote_copy(
            src_ref=stage_b1.at[d - 1],
            dst_ref=zcomm.at[zi],
            send_sem=send_b1.at[d - 1],
            recv_sem=recv_b1.at[zi],
            device_id=(tgt,),
            device_id_type=_MESH,
        )
        rdma.start()
        b1.append(rdma)
    for d in range(1, NZ):
        sz = lax.rem(zi + d, NZ)
        rw = pltpu.make_async_remote_copy(
            src_ref=zcomm.at[sz],
            dst_ref=zcomm.at[sz],
            send_sem=send_b1.at[d - 1],
            recv_sem=recv_b1.at[sz],
            device_id=(my,),
            device_id_type=_MESH,
        )
        rw.wait_recv()
    myrows = pl.ds(slab + zi * SUB, SUB)
    acc = out_ref[myrows, :]
    for d in range(1, NZ):
        sz = lax.rem(zi + d, NZ)
        acc = acc + zcomm[sz].astype(_F32)
    out_ref[myrows, :] = acc
    stage_b2[...] = acc.astype(_BF)
    for rdma in b1:
        rdma.wait_send()

    b2 = []
    for d in range(1, NZ):
        tz = lax.rem(zi + d, NZ)
        tgt = tz * PLANE + q
        rdma = pltpu.make_async_remote_copy(
            src_ref=stage_b2,
            dst_ref=zbcast.at[zi],
            send_sem=send_b2.at[d - 1],
            recv_sem=recv_b2.at[zi],
            device_id=(tgt,),
            device_id_type=_MESH,
        )
        rdma.start()
        b2.append(rdma)
    for d in range(1, NZ):
        sz = lax.rem(zi + d, NZ)
        rw = pltpu.make_async_remote_copy(
            src_ref=zbcast.at[sz],
            dst_ref=zbcast.at[sz],
            send_sem=send_b2.at[d - 1],
            recv_sem=recv_b2.at[sz],
            device_id=(my,),
            device_id_type=_MESH,
        )
        rw.wait_recv()
        out_ref[pl.ds(slab + sz * SUB, SUB), :] = zbcast[sz].astype(_F32)
    for rdma in b2:
        rdma.wait_send()

    stage_cw[...] = out_ref[pl.ds(slab, CHUNK), :].astype(_BF)
    for s in range(4):
        rdma_cw = pltpu.make_async_remote_copy(
            src_ref=stage_cw if s == 0 else p3cw.at[s - 1],
            dst_ref=p3cw.at[s],
            send_sem=send_p3cw,
            recv_sem=recv_p3cw.at[s],
            device_id=(pright,),
            device_id_type=_MESH,
        )
        rdma_cw.start()
        if s < 3:
            rdma_ccw = pltpu.make_async_remote_copy(
                src_ref=stage_cw if s == 0 else p3ccw.at[s - 1],
                dst_ref=p3ccw.at[s],
                send_sem=send_p3ccw,
                recv_sem=recv_p3ccw.at[s],
                device_id=(pleft,),
                device_id_type=_MESH,
            )
            rdma_ccw.start()
        rdma_cw.wait()
        rows = pl.ds(lax.rem(q + PLANE - s, PLANE) * CHUNK, CHUNK)
        out_ref[rows, :] = p3cw[s].astype(_F32)
        if s < 3:
            rdma_ccw.wait()
            rows = pl.ds(lax.rem(q + 2 + s, PLANE) * CHUNK, CHUNK)
            out_ref[rows, :] = p3ccw[s].astype(_F32)


def _ring_allreduce(qh, kh, vh, wo_bf, partial0):
    return pl.pallas_call(
        _allreduce_body,
        out_shape=jax.ShapeDtypeStruct((DM, DM), jnp.float32),
        in_specs=[pl.BlockSpec(memory_space=pltpu.VMEM)] * 5,
        out_specs=pl.BlockSpec(memory_space=pltpu.VMEM),
        scratch_shapes=[
            pltpu.VMEM((CHUNK, DM), _BF),
            pltpu.VMEM((4, CHUNK, DM), _BF),
            pltpu.VMEM((3, CHUNK, DM), _BF),
            pltpu.VMEM((CHUNK, DM), _BF),
            pltpu.VMEM((CHUNK, DM), _BF),
            pltpu.VMEM((NZ, SUB, DM), _BF),
            pltpu.VMEM((NZ, SUB, DM), _BF),
            pltpu.VMEM((NZ - 1, SUB, DM), _BF),
            pltpu.VMEM((SUB, DM), _BF),
            pltpu.VMEM((4, CHUNK, DM), _BF),
            pltpu.VMEM((3, CHUNK, DM), _BF),
            pltpu.SemaphoreType.DMA,
            pltpu.SemaphoreType.DMA((4,)),
            pltpu.SemaphoreType.DMA,
            pltpu.SemaphoreType.DMA((3,)),
            pltpu.SemaphoreType.DMA((NZ - 1,)),
            pltpu.SemaphoreType.DMA((NZ,)),
            pltpu.SemaphoreType.DMA((NZ - 1,)),
            pltpu.SemaphoreType.DMA((NZ,)),
            pltpu.SemaphoreType.DMA,
            pltpu.SemaphoreType.DMA((4,)),
            pltpu.SemaphoreType.DMA,
            pltpu.SemaphoreType.DMA((3,)),
        ],
        compiler_params=pltpu.CompilerParams(collective_id=0),
    )(qh, kh, vh, wo_bf, partial0)


def kernel(x, Wq, K_ext, V_ext, Wo):
    i = lax.axis_index("i")
    hs = Wq.shape[1] // DH
    scale = 0.08838834764831843

    xb = x[0].astype(jnp.bfloat16)
    qall = jnp.dot(xb, Wq.astype(jnp.bfloat16),
                   preferred_element_type=jnp.float32)
    qall = qall.reshape(SQ, hs, DH).astype(jnp.bfloat16)

    k = lax.dynamic_slice_in_dim(K_ext[0], i * hs, hs, axis=1)
    v = lax.dynamic_slice_in_dim(V_ext[0], i * hs, hs, axis=1)
    k = k.astype(jnp.bfloat16)
    v = v.astype(jnp.bfloat16)

    G = 32
    s0w = jnp.einsum("ihd,jhd->hij", qall[:CHUNK], k[:384],
                     preferred_element_type=jnp.float32) * scale
    qi = jnp.arange(CHUNK)[:, None]
    ki = jnp.arange(384)[None, :]
    live = (jnp.abs(qi - ki) <= 128) | (ki < G)
    s0w = jnp.where(live[None], s0w, -1e9)
    w0w = jnp.exp(s0w)
    w0w = w0w / w0w.sum(axis=-1, keepdims=True)
    ctx_b0 = jnp.einsum("hij,jhd->ihd", w0w.astype(jnp.bfloat16), v[:384],
                        preferred_element_type=jnp.float32)

    sg = jnp.einsum("ihd,jhd->hij", qall[:G], k,
                    preferred_element_type=jnp.float32) * scale
    wg = jnp.exp(sg)
    wg = wg / wg.sum(axis=-1, keepdims=True)
    ctxg = jnp.einsum("hij,jhd->ihd", wg.astype(jnp.bfloat16), v,
                      preferred_element_type=jnp.float32)

    ctx0 = jnp.concatenate([ctxg, ctx_b0[G:]], axis=0)
    ctx0 = ctx0.reshape(CHUNK, hs * DH).astype(jnp.bfloat16)
    wo_bf = Wo.astype(jnp.bfloat16)
    partial0 = jnp.dot(ctx0, wo_bf,
                       preferred_element_type=jnp.float32)

    qh = jnp.transpose(qall, (1, 0, 2))
    kh = jnp.transpose(k, (1, 0, 2))
    vh = jnp.transpose(v, (1, 0, 2))

    out = _ring_allreduce(qh, kh, vh, wo_bf, partial0)
    return out.reshape(1, SQ, DM)
